# baseline (speedup 1.0000x reference)
import jax
import jax.numpy as jnp
from jax import lax
from jax.experimental import pallas as pl
from jax.experimental.pallas import tpu as pltpu
from jax.experimental.pallas import tpu_sc as plsc

VOCAB = 1_000_000
D = 32
SEQ = 200
BATCH = 4096

NC, NS, L = 2, 16, 16
NW = NC * NS
ROWS_PER_W = BATCH // NW
RCHUNK = 8
NCHUNK = ROWS_PER_W // RCHUNK


def _tpe_kernel(tok_table, tokens, pos_hbm, out_hbm, idx_v, rows_v, pos_v, sem):
    wid = lax.axis_index("s") * NC + lax.axis_index("c")
    base = wid * ROWS_PER_W

    pltpu.sync_copy(pos_hbm, pos_v)

    @pl.loop(0, NCHUNK)
    def _chunk(i):
        r0 = base + i * RCHUNK
        pltpu.sync_copy(tokens.at[pl.ds(r0, RCHUNK)], idx_v)
        for j in range(RCHUNK):
            pltpu.async_copy(tok_table.at[idx_v.at[j]], rows_v.at[j], sem)
        for j in range(RCHUNK):
            pltpu.make_async_copy(tok_table.at[idx_v.at[j]], rows_v.at[j], sem).wait()

        @pl.loop(0, SEQ)
        def _pos(l):
            p0 = pos_v[l, pl.ds(0, L)]
            p1 = pos_v[l, pl.ds(L, L)]
            for j in range(RCHUNK):
                plsc.addupdate(rows_v.at[j, l, pl.ds(0, L)], p0)
                plsc.addupdate(rows_v.at[j, l, pl.ds(L, L)], p1)

        pltpu.sync_copy(rows_v, out_hbm.at[pl.ds(r0, RCHUNK)])


def kernel(tokens, token_table, position_table):
    mesh = plsc.VectorSubcoreMesh(core_axis_name="c", subcore_axis_name="s")
    run = pl.kernel(
        _tpe_kernel,
        out_type=jax.ShapeDtypeStruct((BATCH, SEQ, D), jnp.float32),
        mesh=mesh,
        scratch_types=[
            pltpu.VMEM((RCHUNK, SEQ), jnp.int32),
            pltpu.VMEM((RCHUNK, SEQ, D), jnp.float32),
            pltpu.VMEM((SEQ, D), jnp.float32),
            pltpu.SemaphoreType.DMA,
        ],
        compiler_params=pltpu.CompilerParams(use_tc_tiling_on_sc=False),
    )
    return run(token_table, tokens.astype(jnp.int32), position_table)
